# Initial kernel scaffold; baseline (speedup 1.0000x reference)
import jax
import jax.numpy as jnp
from jax import lax
from jax.experimental import pallas as pl
from jax.experimental.pallas import tpu as pltpu
from jax.experimental.pallas import tpu_sc as plsc

_COLS = 22
_VOCAB = 12
_D = 128
_B = 4096
_REG = 0.01

_UPAD = 16
_NROWS = _COLS * _UPAD
_NCOLS = _COLS * _VOCAB
_TABLE_WORDS = _NROWS * _NCOLS

_NC = 2
_NS = 16
_LANES = 16
_NW = _NC * _NS
_BPW = _B // _NW
_GROUPS = _BPW // _LANES


def _table_kernel(wp_ref, bp_ref, embp_ref, e_ref, fc_ref, t_ref, sq_ref):
    wp = wp_ref[0, 0]
    bp = bp_ref[0, 0]
    e = e_ref[...]
    tt = e * wp + bp
    sq_ref[...] = jnp.sum(e * e, axis=1, keepdims=True)

    r_ids = lax.broadcasted_iota(jnp.int32, (_NCOLS, _COLS), 0) // _VOCAB
    c_ids = lax.broadcasted_iota(jnp.int32, (_NCOLS, _COLS), 1)
    rmat = (r_ids == c_ids).astype(jnp.float32)

    for i in range(_COLS):
        w = fc_ref[i]
        c = jnp.maximum(
            jnp.sqrt(jnp.sum(w * w, axis=1, keepdims=True)), 1.0)
        wc = w / c
        rep = lax.dot_general(
            rmat, wc, (((1,), (0,)), ((), ())),
            preferred_element_type=jnp.float32)
        cm = tt * rep
        tti = embp_ref[i] * wp + bp
        blk = lax.dot_general(
            tti, cm, (((1,), (1,)), ((), ())),
            preferred_element_type=jnp.float32)
        t_ref[pl.ds(i * _UPAD, _UPAD), :] = blk


def _sc_kernel(table_hbm, feat_hbm, sq_hbm, out_hbm, part_hbm,
               table_v, feat_v, sq_v, out_v, acc_v, sem):
    wid = lax.axis_index("s") * _NC + lax.axis_index("c")
    base = wid * _BPW

    pltpu.async_copy(table_hbm, table_v, sem).wait()
    for i in range(_COLS):
        pltpu.sync_copy(feat_hbm.at[i, pl.ds(base, _BPW)], feat_v.at[i])
    pltpu.sync_copy(sq_hbm, sq_v)

    zeros = jnp.zeros((_LANES,), jnp.float32)
    for i in range(_COLS):
        acc_v[i, :] = zeros

    def group_body(g, carry):
        g16 = g * _LANES
        fj2 = [feat_v[i, pl.ds(g16, _LANES)] + (_VOCAB * i)
               for i in range(_COLS)]
        out_acc = jnp.zeros((_LANES,), jnp.float32)
        for i in range(_COLS):
            hi = fj2[i] * _NCOLS + ((_UPAD - _VOCAB) * _NCOLS * i)
            sqg = plsc.load_gather(sq_v, [fj2[i]])
            plsc.addupdate(acc_v.at[i], sqg)
            for j in range(_COLS):
                idx = hi + fj2[j]
                out_acc = out_acc + plsc.load_gather(table_v, [idx])
        out_v[pl.ds(g16, _LANES)] = out_acc
        return carry

    lax.fori_loop(0, _GROUPS, group_body, 0)

    pltpu.sync_copy(out_v, out_hbm.at[pl.ds(base, _BPW)])
    pltpu.sync_copy(acc_v, part_hbm.at[wid])


def _regs_kernel(p_ref, out_ref):
    x = p_ref[...]
    r = jnp.sum(x, axis=1, keepdims=True)
    row = lax.broadcasted_iota(jnp.int32, (_NW * _COLS, 1), 0) % _COLS
    total = jnp.float32(0.0)
    for i in range(_COLS):
        s_i = jnp.sum(jnp.where(row == i, r, 0.0))
        total = total + jnp.sqrt(s_i)
    out_ref[0, 0] = total * jnp.float32(2 * _COLS * _REG)


def kernel(features, emb, fc_w, w_p, b_p):
    embp = jnp.pad(emb, ((0, 0), (0, _UPAD - _VOCAB), (0, 0)))
    e2 = emb.reshape(_NCOLS, _D)
    fc3 = fc_w.reshape(_COLS, _COLS, _D)
    wp2 = jnp.reshape(w_p, (1, 1))
    bp2 = jnp.reshape(b_p, (1, 1))

    table, sq = pl.pallas_call(
        _table_kernel,
        in_specs=[
            pl.BlockSpec(memory_space=pltpu.SMEM),
            pl.BlockSpec(memory_space=pltpu.SMEM),
            pl.BlockSpec(memory_space=pltpu.ANY),
            pl.BlockSpec(memory_space=pltpu.ANY),
            pl.BlockSpec(memory_space=pltpu.ANY),
        ],
        out_shape=[
            jax.ShapeDtypeStruct((_NROWS, _NCOLS), jnp.float32),
            jax.ShapeDtypeStruct((_NCOLS, 1), jnp.float32),
        ],
    )(wp2, bp2, embp, e2, fc3)

    mesh = plsc.VectorSubcoreMesh(core_axis_name="c", subcore_axis_name="s")
    sc_fn = pl.kernel(
        _sc_kernel,
        out_type=[
            jax.ShapeDtypeStruct((_B,), jnp.float32),
            jax.ShapeDtypeStruct((_NW, _COLS, _LANES), jnp.float32),
        ],
        mesh=mesh,
        scratch_types=[
            pltpu.VMEM((_TABLE_WORDS,), jnp.float32),
            pltpu.VMEM((_COLS, _BPW), jnp.int32),
            pltpu.VMEM((_NCOLS,), jnp.float32),
            pltpu.VMEM((_BPW,), jnp.float32),
            pltpu.VMEM((_COLS, _LANES), jnp.float32),
            pltpu.SemaphoreType.DMA,
        ],
    )
    out_flat, part = sc_fn(table.reshape(_TABLE_WORDS), features,
                           sq.reshape(_NCOLS))

    regs2 = pl.pallas_call(
        _regs_kernel,
        out_shape=jax.ShapeDtypeStruct((1, 1), jnp.float32),
        out_specs=pl.BlockSpec(memory_space=pltpu.SMEM),
    )(part.reshape(_NW * _COLS, _LANES))

    return out_flat.reshape(_B, 1), jnp.reshape(regs2, ())


# speedup vs baseline: 31.5117x; 31.5117x over previous
import jax
import jax.numpy as jnp
from jax import lax
from jax.experimental import pallas as pl
from jax.experimental.pallas import tpu as pltpu
from jax.experimental.pallas import tpu_sc as plsc

_COLS = 22
_VOCAB = 12
_D = 128
_B = 4096
_REG = 0.01

_UPAD = 16
_NROWS = _COLS * _UPAD
_NCOLS = _COLS * _VOCAB
_TABLE_WORDS = _NROWS * _NCOLS

_NC = 2
_NS = 16
_LANES = 16
_NW = _NC * _NS
_BPW = _B // _NW
_GROUPS = _BPW // _LANES


def _table_kernel(wp_ref, bp_ref, embp_ref, e_ref, fc_ref, t_ref, sq_ref):
    wp = wp_ref[0, 0]
    bp = bp_ref[0, 0]
    e = e_ref[...]
    tt = e * wp + bp
    sq_ref[...] = jnp.sum(e * e, axis=1, keepdims=True)

    r_ids = lax.broadcasted_iota(jnp.int32, (_NCOLS, _COLS), 0) // _VOCAB
    c_ids = lax.broadcasted_iota(jnp.int32, (_NCOLS, _COLS), 1)
    rmat = (r_ids == c_ids).astype(jnp.float32)

    for i in range(_COLS):
        w = fc_ref[i]
        c = jnp.maximum(
            jnp.sqrt(jnp.sum(w * w, axis=1, keepdims=True)), 1.0)
        wc = w / c
        rep = lax.dot_general(
            rmat, wc, (((1,), (0,)), ((), ())),
            preferred_element_type=jnp.float32)
        cm = tt * rep
        tti = embp_ref[i] * wp + bp
        blk = lax.dot_general(
            tti, cm, (((1,), (1,)), ((), ())),
            preferred_element_type=jnp.float32)
        t_ref[pl.ds(i * _UPAD, _UPAD), :] = blk


def _sc_kernel(table_hbm, feat_hbm, sq_hbm, out_hbm, part_hbm,
               table_v, feat_v, sq_v, out_v, acc_v, sem):
    wid = lax.axis_index("s") * _NC + lax.axis_index("c")
    base = wid * _BPW

    pltpu.async_copy(table_hbm, table_v, sem).wait()
    for i in range(_COLS):
        pltpu.sync_copy(feat_hbm.at[i, pl.ds(base, _BPW)], feat_v.at[i])
    pltpu.sync_copy(sq_hbm, sq_v)

    zeros = jnp.zeros((_LANES,), jnp.float32)
    for i in range(_COLS):
        acc_v[i, :] = zeros

    def group_body(g, carry):
        g16 = g * _LANES
        fj2 = [feat_v[i, pl.ds(g16, _LANES)] + (_VOCAB * i)
               for i in range(_COLS)]
        out_acc = jnp.zeros((_LANES,), jnp.float32)
        for i in range(_COLS):
            hi = fj2[i] * _NCOLS + ((_UPAD - _VOCAB) * _NCOLS * i)
            sqg = plsc.load_gather(sq_v, [fj2[i]])
            plsc.addupdate(acc_v.at[i], sqg)
            for j in range(_COLS):
                idx = hi + fj2[j]
                out_acc = out_acc + plsc.load_gather(table_v, [idx])
        out_v[pl.ds(g16, _LANES)] = out_acc
        return carry

    lax.fori_loop(0, _GROUPS, group_body, 0)

    pltpu.sync_copy(out_v, out_hbm.at[pl.ds(base, _BPW)])
    pltpu.sync_copy(acc_v, part_hbm.at[wid])


def _regs_kernel(p_ref, out_ref):
    x = p_ref[...]
    r = jnp.sum(x, axis=1, keepdims=True)
    row = lax.broadcasted_iota(jnp.int32, (_NW * _COLS, 1), 0) % _COLS
    total = jnp.float32(0.0)
    for i in range(_COLS):
        s_i = jnp.sum(jnp.where(row == i, r, 0.0))
        total = total + jnp.sqrt(s_i)
    out_ref[0, 0] = total * jnp.float32(2 * _COLS * _REG)


def kernel(features, emb, fc_w, w_p, b_p):
    embp = jnp.pad(emb, ((0, 0), (0, _UPAD - _VOCAB), (0, 0)))
    e2 = emb.reshape(_NCOLS, _D)
    fc3 = fc_w.reshape(_COLS, _COLS, _D)
    wp2 = jnp.reshape(w_p, (1, 1))
    bp2 = jnp.reshape(b_p, (1, 1))

    table, sq = pl.pallas_call(
        _table_kernel,
        in_specs=[
            pl.BlockSpec(memory_space=pltpu.SMEM),
            pl.BlockSpec(memory_space=pltpu.SMEM),
            pl.BlockSpec(memory_space=pltpu.VMEM),
            pl.BlockSpec(memory_space=pltpu.VMEM),
            pl.BlockSpec(memory_space=pltpu.VMEM),
        ],
        out_shape=[
            jax.ShapeDtypeStruct((_NROWS, _NCOLS), jnp.float32),
            jax.ShapeDtypeStruct((_NCOLS, 1), jnp.float32),
        ],
    )(wp2, bp2, embp, e2, fc3)

    mesh = plsc.VectorSubcoreMesh(
        core_axis_name="c", subcore_axis_name="s",
        num_cores=_NC, num_subcores=_NS)
    sc_fn = pl.kernel(
        _sc_kernel,
        out_type=[
            jax.ShapeDtypeStruct((_B,), jnp.float32),
            jax.ShapeDtypeStruct((_NW, _COLS, _LANES), jnp.float32),
        ],
        mesh=mesh,
        compiler_params=pltpu.CompilerParams(needs_layout_passes=False),
        scratch_types=[
            pltpu.VMEM((_TABLE_WORDS,), jnp.float32),
            pltpu.VMEM((_COLS, _BPW), jnp.int32),
            pltpu.VMEM((_NCOLS,), jnp.float32),
            pltpu.VMEM((_BPW,), jnp.float32),
            pltpu.VMEM((_COLS, _LANES), jnp.float32),
            pltpu.SemaphoreType.DMA,
        ],
    )
    out_flat, part = sc_fn(table.reshape(_TABLE_WORDS), features,
                           sq.reshape(_NCOLS))

    regs2 = pl.pallas_call(
        _regs_kernel,
        out_shape=jax.ShapeDtypeStruct((1, 1), jnp.float32),
        out_specs=pl.BlockSpec(memory_space=pltpu.SMEM),
    )(part.reshape(_NW * _COLS, _LANES))

    return out_flat.reshape(_B, 1), jnp.reshape(regs2, ())


# speedup vs baseline: 42.5483x; 1.3502x over previous
import jax
import jax.numpy as jnp
from jax import lax
from jax.experimental import pallas as pl
from jax.experimental.pallas import tpu as pltpu
from jax.experimental.pallas import tpu_sc as plsc

_COLS = 22
_VOCAB = 12
_D = 128
_B = 4096
_REG = 0.01

_UPAD = 16
_NROWS = _COLS * _UPAD
_NCOLS = _COLS * _VOCAB
_TABLE_WORDS = _NROWS * _NCOLS

_NC = 2
_NS = 16
_LANES = 16
_NW = _NC * _NS
_BPW = _B // _NW
_GROUPS = _BPW // _LANES
_NACC = 4


def _table_kernel(wp_ref, bp_ref, embp_ref, e_ref, fc_ref, fct_ref,
                  t_ref, sq_ref):
    wp = wp_ref[0, 0]
    bp = bp_ref[0, 0]
    e = e_ref[...]
    tt = e * wp + bp
    sq_ref[...] = jnp.sum(e * e, axis=1, keepdims=True)

    r_ids = lax.broadcasted_iota(jnp.int32, (_NCOLS, _COLS), 0) // _VOCAB
    c_ids = lax.broadcasted_iota(jnp.int32, (_NCOLS, _COLS), 1)
    rmat = (r_ids == c_ids).astype(jnp.float32)
    j_ids = lax.broadcasted_iota(jnp.int32, (_COLS, 1), 0)

    def _constrain(w):
        c = jnp.maximum(jnp.sqrt(jnp.sum(w * w, axis=1, keepdims=True)), 1.0)
        return w / c

    for i in range(_COLS):
        wc_ij = _constrain(fc_ref[i])
        wc_ji = _constrain(fct_ref[i])
        wsym = wc_ij + jnp.where(j_ids == i, 0.0, wc_ji)
        rep = lax.dot_general(
            rmat, wsym, (((1,), (0,)), ((), ())),
            preferred_element_type=jnp.float32)
        cm = tt * rep
        tti = embp_ref[i] * wp + bp
        blk = lax.dot_general(
            tti, cm, (((1,), (1,)), ((), ())),
            preferred_element_type=jnp.float32)
        t_ref[pl.ds(i * _UPAD, _UPAD), :] = blk


def _sc_kernel(table_hbm, feat_hbm, sq_hbm, out_hbm, part_hbm,
               table_v, feat_v, sq_v, out_v, acc_v, sem):
    wid = lax.axis_index("s") * _NC + lax.axis_index("c")
    base = wid * _BPW

    cp = pltpu.async_copy(table_hbm, table_v, sem)
    pltpu.sync_copy(feat_hbm.at[:, pl.ds(base, _BPW)], feat_v)
    pltpu.sync_copy(sq_hbm, sq_v)
    zeros = jnp.zeros((_LANES,), jnp.float32)
    for i in range(_COLS):
        acc_v[pl.ds(i * _LANES, _LANES)] = zeros
    cp.wait()

    def group_body(g, carry):
        g16 = g * _LANES
        fj2 = [feat_v[i, pl.ds(g16, _LANES)] + (_VOCAB * i)
               for i in range(_COLS)]
        accs = [jnp.zeros((_LANES,), jnp.float32) for _ in range(_NACC)]
        n = 0
        for i in range(_COLS):
            hi = fj2[i] * _NCOLS + ((_UPAD - _VOCAB) * _NCOLS * i)
            sqg = plsc.load_gather(sq_v, [fj2[i]])
            plsc.addupdate(acc_v.at[pl.ds(i * _LANES, _LANES)], sqg)
            for j in range(i, _COLS):
                idx = hi + fj2[j]
                accs[n % _NACC] = accs[n % _NACC] + plsc.load_gather(
                    table_v, [idx])
                n += 1
        out_v[pl.ds(g16, _LANES)] = (
            (accs[0] + accs[1]) + (accs[2] + accs[3]))
        return carry

    lax.fori_loop(0, _GROUPS, group_body, 0)

    pltpu.sync_copy(out_v, out_hbm.at[pl.ds(base, _BPW)])
    pltpu.sync_copy(acc_v, part_hbm.at[wid])


def _regs_kernel(p_ref, out_ref):
    x = p_ref[...]
    s = jnp.sum(x, axis=0, keepdims=True)
    g_ids = lax.broadcasted_iota(jnp.int32, (_NROWS, _COLS), 0) // _LANES
    c_ids = lax.broadcasted_iota(jnp.int32, (_NROWS, _COLS), 1)
    gmat = (g_ids == c_ids).astype(jnp.float32)
    per_col = lax.dot_general(
        s, gmat, (((1,), (0,)), ((), ())),
        preferred_element_type=jnp.float32)
    out_ref[0, 0] = jnp.sum(jnp.sqrt(per_col)) * jnp.float32(
        2 * _COLS * _REG)


def kernel(features, emb, fc_w, w_p, b_p):
    embp = jnp.pad(emb, ((0, 0), (0, _UPAD - _VOCAB), (0, 0)))
    e2 = emb.reshape(_NCOLS, _D)
    fc3 = fc_w.reshape(_COLS, _COLS, _D)
    fct3 = fc3.transpose(1, 0, 2)
    wp2 = jnp.reshape(w_p, (1, 1))
    bp2 = jnp.reshape(b_p, (1, 1))

    table, sq = pl.pallas_call(
        _table_kernel,
        in_specs=[
            pl.BlockSpec(memory_space=pltpu.SMEM),
            pl.BlockSpec(memory_space=pltpu.SMEM),
            pl.BlockSpec(memory_space=pltpu.VMEM),
            pl.BlockSpec(memory_space=pltpu.VMEM),
            pl.BlockSpec(memory_space=pltpu.VMEM),
            pl.BlockSpec(memory_space=pltpu.VMEM),
        ],
        out_shape=[
            jax.ShapeDtypeStruct((_NROWS, _NCOLS), jnp.float32),
            jax.ShapeDtypeStruct((_NCOLS, 1), jnp.float32),
        ],
    )(wp2, bp2, embp, e2, fc3, fct3)

    mesh = plsc.VectorSubcoreMesh(
        core_axis_name="c", subcore_axis_name="s",
        num_cores=_NC, num_subcores=_NS)
    sc_fn = pl.kernel(
        _sc_kernel,
        out_type=[
            jax.ShapeDtypeStruct((_B,), jnp.float32),
            jax.ShapeDtypeStruct((_NW, _NROWS), jnp.float32),
        ],
        mesh=mesh,
        compiler_params=pltpu.CompilerParams(needs_layout_passes=False),
        scratch_types=[
            pltpu.VMEM((_TABLE_WORDS,), jnp.float32),
            pltpu.VMEM((_COLS, _BPW), jnp.int32),
            pltpu.VMEM((_NCOLS,), jnp.float32),
            pltpu.VMEM((_BPW,), jnp.float32),
            pltpu.VMEM((_NROWS,), jnp.float32),
            pltpu.SemaphoreType.DMA,
        ],
    )
    out_flat, part = sc_fn(table.reshape(_TABLE_WORDS), features,
                           sq.reshape(_NCOLS))

    regs2 = pl.pallas_call(
        _regs_kernel,
        out_shape=jax.ShapeDtypeStruct((1, 1), jnp.float32),
        out_specs=pl.BlockSpec(memory_space=pltpu.SMEM),
    )(part)

    return out_flat.reshape(_B, 1), jnp.reshape(regs2, ())
